# baseline (speedup 1.0000x reference)
import functools

import jax
import jax.numpy as jnp
from jax import lax
from jax.experimental import pallas as pl
from jax.experimental.pallas import tpu as pltpu
from jax.experimental.pallas import tpu_sc as plsc

VOCAB = 1000000
D = 16
P = 16
B = 1024
L = 20
NC = 2
NS = 16
LANES = 16

VTAIL = VOCAB % 128
VMAIN = VOCAB - VTAIL
VPAD = (VOCAB + 127) // 128 * 128
STAGE = 2 * 8 * VPAD

TOK_PER = (B * L) // NS
N_ELEMS = 8 * TOK_PER
N_VREGS = N_ELEMS // LANES
CHUNKS = L * (D // 8) * (B // 128)

SLAB_W = 4096
N_FULL_SLABS = VMAIN // SLAB_W
SLAB_REM = VMAIN - N_FULL_SLABS * SLAB_W

_SIN_C = (3.1415926409395194, -5.16771227680099, 2.550158280611899,
          -0.5992355764431792, 0.08207129109386657, -0.0072673205351405645,
          0.00039296507712438533)
_COS_C = (0.9999999999193584, -4.9348021895543805, 4.0587118821364,
          -1.3352607094469748, 0.23532212897209104, -0.025787854658556375,
          0.0019059119592104157, -8.916973064498901e-05)


def _horner(w, coeffs):
    r = jnp.full((LANES,), coeffs[-1], dtype=jnp.float32)
    for c in coeffs[-2::-1]:
        r = r * w + c
    return r


def _sincos(v):
    a = jnp.abs(v)
    e = jnp.exp(a * -2.0)
    u = jnp.sign(v) * ((1.0 - e) / (1.0 + e))
    w = u * u
    return _horner(w, _COS_C), u * _horner(w, _SIN_C)


def _make_sc_kernel():
    mesh = plsc.VectorSubcoreMesh(core_axis_name="c", subcore_axis_name="s",
                                  num_cores=NC, num_subcores=NS)

    @functools.partial(
        pl.kernel,
        out_type=(jax.ShapeDtypeStruct((STAGE,), jnp.float32),
                  jax.ShapeDtypeStruct((CHUNKS, 8, 128), jnp.float32),
                  jax.ShapeDtypeStruct((CHUNKS, 8, 128), jnp.float32)),
        mesh=mesh,
        scratch_types=[
            pltpu.VMEM((8, SLAB_W), jnp.float32),
            pltpu.VMEM((2 * 8 * VTAIL,), jnp.float32),
            pltpu.VMEM((128 * L,), jnp.int32),
            pltpu.VMEM((N_ELEMS,), jnp.int32),
            pltpu.VMEM((N_ELEMS,), jnp.float32),
            pltpu.VMEM((L // 2, 8, 128), jnp.float32),
            pltpu.VMEM((L // 2, 8, 128), jnp.float32),
            pltpu.SemaphoreType.DMA,
            pltpu.SemaphoreType.DMA,
        ],
        compiler_params=pltpu.CompilerParams(use_tc_tiling_on_sc=True,
                                             needs_layout_passes=False),
    )
    def sc_fn(tok_hbm, tab_hbm, tail_hbm, stage_hbm, re_hbm, im_hbm,
              slab_v, tail_v, tok_v, idxf_v, rows_v, re_v, im_v, sem, sem2):
        core = lax.axis_index("c")
        sid = lax.axis_index("s")
        row0 = core * 8
        srow = core * 8 * VPAD

        def slab_body(t, carry):
            ch = sid + t * NS

            @pl.when(ch < N_FULL_SLABS)
            def _():
                cps = [
                    pltpu.async_copy(
                        tab_hbm.at[row0 + din, pl.ds(ch * SLAB_W, SLAB_W)],
                        stage_hbm.at[pl.ds(srow + din * VPAD + ch * SLAB_W,
                                           SLAB_W)],
                        sem)
                    for din in range(8)
                ]
                for cp in cps:
                    cp.wait()

            @pl.when(ch == N_FULL_SLABS)
            def _():
                cps = [
                    pltpu.async_copy(
                        tab_hbm.at[row0 + din,
                                   pl.ds(N_FULL_SLABS * SLAB_W, SLAB_REM)],
                        stage_hbm.at[pl.ds(
                            srow + din * VPAD + N_FULL_SLABS * SLAB_W,
                            SLAB_REM)],
                        sem)
                    for din in range(8)
                ]
                for cp in cps:
                    cp.wait()

            return carry

        lax.fori_loop(0, (N_FULL_SLABS + NS) // NS, slab_body, 0)

        @pl.when(sid == 0)
        def _():
            pltpu.sync_copy(tail_hbm, tail_v)
            cps = [
                pltpu.async_copy(
                    tail_v.at[pl.ds((core * 8 + din) * VTAIL, VTAIL)],
                    stage_hbm.at[pl.ds(srow + din * VPAD + VMAIN, VTAIL)],
                    sem)
                for din in range(8)
            ]
            for cp in cps:
                cp.wait()

        plsc.subcore_barrier()

        bblk = sid // 2
        lgrp = sid % 2

        pltpu.sync_copy(tok_hbm.at[pl.ds(bblk * 128 * L, 128 * L)], tok_v)

        iota20 = lax.iota(jnp.int32, LANES) * L

        def idx_body(g, carry):
            grp = g & 7
            din = (g >> 3) & 7
            l_off = g >> 6
            pos0 = grp * (LANES * L) + lgrp * (L // 2) + l_off
            tok16 = plsc.load_gather(tok_v, [iota20 + pos0])
            idxf_v[pl.ds(g * LANES, LANES)] = (
                tok16 + (srow + din * VPAD))
            return carry

        lax.fori_loop(0, N_VREGS, idx_body, 0)

        pltpu.async_copy(stage_hbm.at[idxf_v], rows_v, sem2).wait()

        def body(g, carry):
            c16, s16 = _sincos(rows_v[pl.ds(g * LANES, LANES)])
            l_off = g >> 6
            din = (g >> 3) & 7
            col = (g & 7) * LANES
            re_v[l_off, din, pl.ds(col, LANES)] = c16
            im_v[l_off, din, pl.ds(col, LANES)] = s16
            return carry

        lax.fori_loop(0, N_VREGS, body, 0)

        cps = []
        for l_off in range(L // 2):
            lq = lgrp * (L // 2) + l_off
            chunk = (lq * 2) * 8 + bblk
            cps.append(pltpu.async_copy(
                re_v.at[l_off], re_hbm.at[chunk + core * 8], sem2))
            cps.append(pltpu.async_copy(
                im_v.at[l_off], im_hbm.at[chunk + core * 8], sem2))
        for cp in cps:
            cp.wait()

    return sc_fn


def kernel(token_ids, embedding_table):
    tok = token_ids.reshape(B * L).astype(jnp.int32)
    tail = embedding_table[VMAIN:, :].T.reshape(D * VTAIL)
    _, re3, im3 = _make_sc_kernel()(tok, embedding_table.T, tail)

    def planes(x):
        return (x.reshape(L, 2, 8, 8, 128).transpose(2, 4, 0, 1, 3)
                .reshape(B, L, D))

    re = planes(re3)
    im = planes(im3)
    base = lax.complex(re, im).reshape(B, L, 1, D)
    return jnp.broadcast_to(base, (B, L, P, D))


# speedup vs baseline: 5.1959x; 5.1959x over previous
import functools

import jax
import jax.numpy as jnp
from jax import lax
from jax.experimental import pallas as pl
from jax.experimental.pallas import tpu as pltpu
from jax.experimental.pallas import tpu_sc as plsc

VOCAB = 1000000
D = 16
P = 16
B = 1024
L = 20
NC = 2
NS = 16
LANES = 16

VTAIL = VOCAB % 128
VMAIN = VOCAB - VTAIL
VPAD = (VOCAB + 127) // 128 * 128
STAGE = 2 * 8 * VPAD

TOK_PER = (B * L) // NS
N_ELEMS = 8 * TOK_PER
N_VREGS = N_ELEMS // LANES
CHUNKS = L * (D // 8) * (B // 128)

SLAB_W = 4096
N_FULL_SLABS = VMAIN // SLAB_W
SLAB_REM = VMAIN - N_FULL_SLABS * SLAB_W

_SIN_C = (3.1415926409395194, -5.16771227680099, 2.550158280611899,
          -0.5992355764431792, 0.08207129109386657, -0.0072673205351405645,
          0.00039296507712438533)
_COS_C = (0.9999999999193584, -4.9348021895543805, 4.0587118821364,
          -1.3352607094469748, 0.23532212897209104, -0.025787854658556375,
          0.0019059119592104157, -8.916973064498901e-05)


def _horner(w, coeffs):
    r = jnp.full((LANES,), coeffs[-1], dtype=jnp.float32)
    for c in coeffs[-2::-1]:
        r = r * w + c
    return r


def _sincos(v):
    a = jnp.abs(v)
    e = jnp.exp(a * -2.0)
    u = jnp.sign(v) * ((1.0 - e) / (1.0 + e))
    w = u * u
    return _horner(w, _COS_C), u * _horner(w, _SIN_C)


def _make_sc_kernel():
    mesh = plsc.VectorSubcoreMesh(core_axis_name="c", subcore_axis_name="s",
                                  num_cores=NC, num_subcores=NS)

    @functools.partial(
        pl.kernel,
        out_type=(jax.ShapeDtypeStruct((STAGE,), jnp.float32),
                  jax.ShapeDtypeStruct((CHUNKS, 8, 128), jnp.float32),
                  jax.ShapeDtypeStruct((CHUNKS, 8, 128), jnp.float32)),
        mesh=mesh,
        scratch_types=[
            pltpu.VMEM((8, SLAB_W), jnp.float32),
            pltpu.VMEM((2 * 8 * VTAIL,), jnp.float32),
            pltpu.VMEM((128 * L,), jnp.int32),
            pltpu.VMEM((N_ELEMS,), jnp.int32),
            pltpu.VMEM((N_ELEMS,), jnp.float32),
            pltpu.VMEM((L // 2, 8, 128), jnp.float32),
            pltpu.VMEM((L // 2, 8, 128), jnp.float32),
            pltpu.SemaphoreType.DMA,
            pltpu.SemaphoreType.DMA,
        ],
        compiler_params=pltpu.CompilerParams(use_tc_tiling_on_sc=True,
                                             needs_layout_passes=False),
    )
    def sc_fn(tok_hbm, tab_hbm, tail_hbm, stage_hbm, re_hbm, im_hbm,
              slab_v, tail_v, tok_v, idxf_v, rows_v, re_v, im_v, sem, sem2):
        core = lax.axis_index("c")
        sid = lax.axis_index("s")
        row0 = core * 8
        srow = core * 8 * VPAD

        def slab_body(t, carry):
            ch = sid + t * NS

            @pl.when(ch < N_FULL_SLABS)
            def _():
                pltpu.sync_copy(
                    tab_hbm.at[pl.ds(row0, 8), pl.ds(ch * SLAB_W, SLAB_W)],
                    slab_v)
                cps = [
                    pltpu.async_copy(
                        slab_v.at[din],
                        stage_hbm.at[pl.ds(srow + din * VPAD + ch * SLAB_W,
                                           SLAB_W)],
                        sem)
                    for din in range(8)
                ]
                for cp in cps:
                    cp.wait()

            @pl.when(ch == N_FULL_SLABS)
            def _():
                pltpu.sync_copy(
                    tab_hbm.at[pl.ds(row0, 8),
                               pl.ds(N_FULL_SLABS * SLAB_W, SLAB_REM)],
                    slab_v.at[:, pl.ds(0, SLAB_REM)])
                cps = [
                    pltpu.async_copy(
                        slab_v.at[din, pl.ds(0, SLAB_REM)],
                        stage_hbm.at[pl.ds(
                            srow + din * VPAD + N_FULL_SLABS * SLAB_W,
                            SLAB_REM)],
                        sem)
                    for din in range(8)
                ]
                for cp in cps:
                    cp.wait()

            return carry

        lax.fori_loop(0, (N_FULL_SLABS + NS) // NS, slab_body, 0)

        @pl.when(sid == 0)
        def _():
            pltpu.sync_copy(tail_hbm, tail_v)
            cps = [
                pltpu.async_copy(
                    tail_v.at[pl.ds((core * 8 + din) * VTAIL, VTAIL)],
                    stage_hbm.at[pl.ds(srow + din * VPAD + VMAIN, VTAIL)],
                    sem)
                for din in range(8)
            ]
            for cp in cps:
                cp.wait()

        plsc.subcore_barrier()

        bblk = sid // 2
        lgrp = sid % 2

        pltpu.sync_copy(tok_hbm.at[pl.ds(bblk * 128 * L, 128 * L)], tok_v)

        iota20 = lax.iota(jnp.int32, LANES) * L

        def idx_body(g, carry):
            grp = g & 7
            din = (g >> 3) & 7
            l_off = g >> 6
            pos0 = grp * (LANES * L) + lgrp * (L // 2) + l_off
            tok16 = plsc.load_gather(tok_v, [iota20 + pos0])
            idxf_v[pl.ds(g * LANES, LANES)] = (
                tok16 + (srow + din * VPAD))
            return carry

        lax.fori_loop(0, N_VREGS, idx_body, 0)

        pltpu.async_copy(stage_hbm.at[idxf_v], rows_v, sem2).wait()

        def body(g, carry):
            c16, s16 = _sincos(rows_v[pl.ds(g * LANES, LANES)])
            l_off = g >> 6
            din = (g >> 3) & 7
            col = (g & 7) * LANES
            re_v[l_off, din, pl.ds(col, LANES)] = c16
            im_v[l_off, din, pl.ds(col, LANES)] = s16
            return carry

        lax.fori_loop(0, N_VREGS, body, 0)

        cps = []
        for l_off in range(L // 2):
            lq = lgrp * (L // 2) + l_off
            chunk = (lq * 2) * 8 + bblk
            cps.append(pltpu.async_copy(
                re_v.at[l_off], re_hbm.at[chunk + core * 8], sem2))
            cps.append(pltpu.async_copy(
                im_v.at[l_off], im_hbm.at[chunk + core * 8], sem2))
        for cp in cps:
            cp.wait()

    return sc_fn


def kernel(token_ids, embedding_table):
    tok = token_ids.reshape(B * L).astype(jnp.int32)
    tail = embedding_table[VMAIN:, :].T.reshape(D * VTAIL)
    _, re3, im3 = _make_sc_kernel()(tok, embedding_table.T, tail)

    def planes(x):
        return (x.reshape(L, 2, 8, 8, 128).transpose(2, 4, 0, 1, 3)
                .reshape(B, L, D))

    re = planes(re3)
    im = planes(im3)
    base = lax.complex(re, im).reshape(B, L, 1, D)
    return jnp.broadcast_to(base, (B, L, P, D))
